# baseline (speedup 1.0000x reference)
import jax
import jax.numpy as jnp
from jax.experimental import pallas as pl
from jax.experimental.pallas import tpu as pltpu

Q = 1024
D = 128
K = 100000
KT = 4096
G = (K + KT - 1) // KT
KP = G * KT
NCH = KT // 128
TAIL = K - (G - 1) * KT
TAIL_PAD = 2048
NCH_T = TAIL_PAD // 128
BIG = 3.0e38


def _semidual_kernel(x_ref, y_ref, psi_ref, out1_ref, out2_ref,
                     acc_ref, xb_ref, psum_ref):
    k = pl.program_id(0)

    @pl.when(k == 0)
    def _init():
        acc_ref[...] = jnp.full((Q, 128), BIG, dtype=jnp.float32)
        xb_ref[...] = (x_ref[...] * -2.0).astype(jnp.bfloat16)
        psum_ref[0, 0] = 0.0

    psi = psi_ref[0]
    ones = jnp.ones((1, D), dtype=jnp.bfloat16)

    @pl.when(k < G - 1)
    def _full():
        yb = y_ref[...].astype(jnp.bfloat16)
        xy = jax.lax.dot_general(
            xb_ref[...], yb, (((1,), (1,)), ((), ())),
            preferred_element_type=jnp.float32,
        )
        y2 = jax.lax.dot_general(
            ones, yb * yb, (((1,), (1,)), ((), ())),
            preferred_element_type=jnp.float32,
        )
        s = xy + (y2 - psi)
        m = acc_ref[...]
        for i in range(NCH):
            m = jnp.minimum(m, s[:, i * 128:(i + 1) * 128])
        acc_ref[...] = m

    @pl.when(k == G - 1)
    def _tail():
        row = jax.lax.broadcasted_iota(jnp.int32, (TAIL_PAD, 1), 0)
        yb = jnp.where(row < TAIL, y_ref[0:TAIL_PAD, :],
                       0.0).astype(jnp.bfloat16)
        xy = jax.lax.dot_general(
            xb_ref[...], yb, (((1,), (1,)), ((), ())),
            preferred_element_type=jnp.float32,
        )
        y2 = jax.lax.dot_general(
            ones, yb * yb, (((1,), (1,)), ((), ())),
            preferred_element_type=jnp.float32,
        )
        col = jax.lax.broadcasted_iota(jnp.int32, (1, TAIL_PAD), 1)
        b = jnp.where(col < TAIL, y2 - psi[:, 0:TAIL_PAD], BIG)
        s = xy + b
        m = acc_ref[...]
        for i in range(NCH_T):
            m = jnp.minimum(m, s[:, i * 128:(i + 1) * 128])

        x = x_ref[...]
        x2 = jnp.sum(x * x, axis=1, keepdims=True)
        mrow = jnp.min(m, axis=1, keepdims=True)
        out1_ref[0, 0] = jnp.sum(mrow + x2) * (1.0 / Q)
        out2_ref[0, 0] = (psum_ref[0, 0] + jnp.sum(psi)) * (1.0 / K)

    @pl.when(k < G - 1)
    def _psum():
        psum_ref[0, 0] += jnp.sum(psi)


@jax.jit
def _semidual(inputx, inputy, psi):
    psi_pad = jnp.pad(psi, (0, KP - K)).reshape(G, 1, KT)
    out1, out2 = pl.pallas_call(
        _semidual_kernel,
        grid=(G,),
        in_specs=[
            pl.BlockSpec((Q, D), lambda k: (0, 0)),
            pl.BlockSpec((KT, D), lambda k: (k, 0)),
            pl.BlockSpec((1, 1, KT), lambda k: (k, 0, 0)),
        ],
        out_specs=[
            pl.BlockSpec(memory_space=pltpu.SMEM),
            pl.BlockSpec(memory_space=pltpu.SMEM),
        ],
        out_shape=[
            jax.ShapeDtypeStruct((1, 1), jnp.float32),
            jax.ShapeDtypeStruct((1, 1), jnp.float32),
        ],
        scratch_shapes=[
            pltpu.VMEM((Q, 128), jnp.float32),
            pltpu.VMEM((Q, D), jnp.bfloat16),
            pltpu.SMEM((1, 1), jnp.float32),
        ],
        compiler_params=pltpu.CompilerParams(
            dimension_semantics=("arbitrary",),
        ),
    )(inputx, inputy, psi_pad)
    return out1[0, 0], out2[0, 0]


def kernel(inputx, inputy, psi):
    return _semidual(inputx, inputy, psi)


# speedup vs baseline: 1.0606x; 1.0606x over previous
import jax
import jax.numpy as jnp
from jax.experimental import pallas as pl
from jax.experimental.pallas import tpu as pltpu

Q = 1024
D = 128
K = 100000
KT = 8192
G = (K + KT - 1) // KT
KP = G * KT
NCH = KT // 128
TAIL = K - (G - 1) * KT
TAIL_PAD = 2048
NCH_T = TAIL_PAD // 128
BIG = 3.0e38


def _semidual_kernel(x_ref, y_ref, psi_ref, out1_ref, out2_ref,
                     acc_ref, xb_ref, psum_ref):
    k = pl.program_id(0)

    @pl.when(k == 0)
    def _init():
        acc_ref[...] = jnp.full((Q, 128), BIG, dtype=jnp.float32)
        xb_ref[...] = x_ref[...] * -2.0
        psum_ref[0, 0] = 0.0

    psi = psi_ref[0]
    ones = jnp.ones((1, D), dtype=jnp.float32)

    @pl.when(k < G - 1)
    def _full():
        yb = y_ref[...]
        xy = jax.lax.dot_general(
            xb_ref[...], yb, (((1,), (1,)), ((), ())),
            preferred_element_type=jnp.float32,
        )
        y2 = jax.lax.dot_general(
            ones, yb * yb, (((1,), (1,)), ((), ())),
            preferred_element_type=jnp.float32,
        )
        s = xy + (y2 - psi)
        m = acc_ref[...]
        for i in range(NCH):
            m = jnp.minimum(m, s[:, i * 128:(i + 1) * 128])
        acc_ref[...] = m

    @pl.when(k == G - 1)
    def _tail():
        row = jax.lax.broadcasted_iota(jnp.int32, (TAIL_PAD, 1), 0)
        yb = jnp.where(row < TAIL, y_ref[0:TAIL_PAD, :], 0.0)
        xy = jax.lax.dot_general(
            xb_ref[...], yb, (((1,), (1,)), ((), ())),
            preferred_element_type=jnp.float32,
        )
        y2 = jax.lax.dot_general(
            ones, yb * yb, (((1,), (1,)), ((), ())),
            preferred_element_type=jnp.float32,
        )
        col = jax.lax.broadcasted_iota(jnp.int32, (1, TAIL_PAD), 1)
        b = jnp.where(col < TAIL, y2 - psi[:, 0:TAIL_PAD], BIG)
        s = xy + b
        m = acc_ref[...]
        for i in range(NCH_T):
            m = jnp.minimum(m, s[:, i * 128:(i + 1) * 128])

        x = x_ref[...]
        x2 = jnp.sum(x * x, axis=1, keepdims=True)
        mrow = jnp.min(m, axis=1, keepdims=True)
        out1_ref[0, 0] = jnp.sum(mrow + x2) * (1.0 / Q)
        out2_ref[0, 0] = (psum_ref[0, 0] + jnp.sum(psi)) * (1.0 / K)

    @pl.when(k < G - 1)
    def _psum():
        psum_ref[0, 0] += jnp.sum(psi)


@jax.jit
def _semidual(inputx, inputy, psi):
    psi_pad = jnp.pad(psi, (0, KP - K)).reshape(G, 1, KT)
    out1, out2 = pl.pallas_call(
        _semidual_kernel,
        grid=(G,),
        in_specs=[
            pl.BlockSpec((Q, D), lambda k: (0, 0)),
            pl.BlockSpec((KT, D), lambda k: (k, 0)),
            pl.BlockSpec((1, 1, KT), lambda k: (k, 0, 0)),
        ],
        out_specs=[
            pl.BlockSpec(memory_space=pltpu.SMEM),
            pl.BlockSpec(memory_space=pltpu.SMEM),
        ],
        out_shape=[
            jax.ShapeDtypeStruct((1, 1), jnp.float32),
            jax.ShapeDtypeStruct((1, 1), jnp.float32),
        ],
        scratch_shapes=[
            pltpu.VMEM((Q, 128), jnp.float32),
            pltpu.VMEM((Q, D), jnp.float32),
            pltpu.SMEM((1, 1), jnp.float32),
        ],
        compiler_params=pltpu.CompilerParams(
            dimension_semantics=("arbitrary",),
        ),
    )(inputx, inputy, psi_pad)
    return out1[0, 0], out2[0, 0]


def kernel(inputx, inputy, psi):
    return _semidual(inputx, inputy, psi)
